# baseline (speedup 1.0000x reference)
import functools

import jax
import jax.numpy as jnp
from jax import lax
from jax.experimental import pallas as pl
from jax.experimental.pallas import tpu as pltpu
from jax.experimental.pallas import tpu_sc as plsc


def _router_body(T, E, n_total, tok_base, x_ref, w1t_ref, b1_ref, w2t_ref,
                 b2_ref, i0_ref, i1_ref, g0_ref, g1_ref):
    i = pl.program_id(0)
    h = jnp.dot(x_ref[...], w1t_ref[...], preferred_element_type=jnp.float32)
    h = jnp.maximum(h + b1_ref[...], 0.0)
    logits = jnp.dot(h, w2t_ref[...], preferred_element_type=jnp.float32)
    logits = logits + b2_ref[...]
    m = jnp.max(logits, axis=1, keepdims=True)
    p = jnp.exp(logits - m)
    p = p / jnp.sum(p, axis=1, keepdims=True)
    lane = lax.broadcasted_iota(jnp.int32, (T, E), 1)
    p1 = jnp.max(p, axis=1, keepdims=True)
    i1 = jnp.min(jnp.where(p == p1, lane, E), axis=1, keepdims=True)
    pm = jnp.where(lane == i1, -jnp.inf, p)
    p2 = jnp.max(pm, axis=1, keepdims=True)
    i2 = jnp.min(jnp.where(pm == p2, lane, E), axis=1, keepdims=True)
    tok = tok_base + i * T + lax.broadcasted_iota(jnp.int32, (T, 1), 0)
    i0_ref[...] = jnp.broadcast_to(i1 * n_total + tok, (T, 16))
    i1_ref[...] = jnp.broadcast_to(i2 * n_total + tok, (T, 16))
    g0_ref[...] = jnp.broadcast_to(p1, (T, 16))
    g1_ref[...] = jnp.broadcast_to(p2, (T, 16))


def _router(x, w1t, b1, w2t, b2, n_total, tok_base, T=1024):
    N, H = x.shape
    E = w2t.shape[1]
    body = functools.partial(_router_body, T, E, n_total, tok_base)
    grid = (N // T,)
    outs = pl.pallas_call(
        body,
        grid=grid,
        in_specs=[
            pl.BlockSpec((T, H), lambda i: (i, 0)),
            pl.BlockSpec((H, H), lambda i: (0, 0)),
            pl.BlockSpec((1, H), lambda i: (0, 0)),
            pl.BlockSpec((H, E), lambda i: (0, 0)),
            pl.BlockSpec((1, E), lambda i: (0, 0)),
        ],
        out_specs=[
            pl.BlockSpec((T, 16), lambda i: (i, 0)),
            pl.BlockSpec((T, 16), lambda i: (i, 0)),
            pl.BlockSpec((T, 16), lambda i: (i, 0)),
            pl.BlockSpec((T, 16), lambda i: (i, 0)),
        ],
        out_shape=[
            jax.ShapeDtypeStruct((N, 16), jnp.int32),
            jax.ShapeDtypeStruct((N, 16), jnp.int32),
            jax.ShapeDtypeStruct((N, 16), jnp.float32),
            jax.ShapeDtypeStruct((N, 16), jnp.float32),
        ],
    )(x, w1t, b1, w2t, b2)
    return outs


def _make_combine(N, H, G=8, n_total=None):
    n_workers = 32
    per_w = N // n_workers
    n_chunks = per_w // G
    assert n_chunks % 4 == 0
    mesh = plsc.VectorSubcoreMesh(
        core_axis_name="c", subcore_axis_name="s", num_cores=2, num_subcores=16)

    @functools.partial(
        pl.kernel,
        out_type=jax.ShapeDtypeStruct((N, H), jnp.float32),
        mesh=mesh,
        scratch_types=[
            pltpu.VMEM((2 * per_w,), jnp.int32),
            pltpu.VMEM((2 * per_w * 16,), jnp.float32),
            pltpu.VMEM((4, 2 * G, H), jnp.float32),
            pltpu.VMEM((4, G, H), jnp.float32),
            pltpu.SemaphoreType.DMA,
            pltpu.SemaphoreType.DMA,
            pltpu.SemaphoreType.DMA,
            pltpu.SemaphoreType.DMA,
            pltpu.SemaphoreType.DMA,
            pltpu.SemaphoreType.DMA,
            pltpu.SemaphoreType.DMA,
            pltpu.SemaphoreType.DMA,
        ],
    )
    def combine(table, i0, i1, g0, g1, out,
                idx_v, g_v, r_v, o_v,
                sg0, sg1, sg2, sg3, so0, so1, so2, so3):
        wid = lax.axis_index("s") * 2 + lax.axis_index("c")
        wbase = wid * per_w
        sg = (sg0, sg1, sg2, sg3)
        so = (so0, so1, so2, so3)

        pltpu.sync_copy(i0.at[pl.ds(wbase, per_w)], idx_v.at[pl.ds(0, per_w)])
        pltpu.sync_copy(i1.at[pl.ds(wbase, per_w)],
                        idx_v.at[pl.ds(per_w, per_w)])
        pltpu.sync_copy(g0.at[pl.ds(wbase * 16, per_w * 16)],
                        g_v.at[pl.ds(0, per_w * 16)])
        pltpu.sync_copy(g1.at[pl.ds(wbase * 16, per_w * 16)],
                        g_v.at[pl.ds(per_w * 16, per_w * 16)])

        def gather_descs(cc, b):
            base = cc * G
            d0 = pltpu.make_async_copy(
                table.at[idx_v.at[pl.ds(base, G)]],
                r_v.at[b, pl.ds(0, G)], sg[b])
            d1 = pltpu.make_async_copy(
                table.at[idx_v.at[pl.ds(per_w + base, G)]],
                r_v.at[b, pl.ds(G, G)], sg[b])
            return d0, d1

        def out_desc(cc, b):
            return pltpu.make_async_copy(
                o_v.at[b], out.at[pl.ds(wbase + cc * G, G)], so[b])

        for pre in range(3):
            d0, d1 = gather_descs(pre, pre)
            d0.start()
            d1.start()

        @pl.loop(0, n_chunks, step=4)
        def _quad(c):
            for b in range(4):
                cc = c + b
                w0, w1 = gather_descs(cc, b)
                w0.wait()
                w1.wait()

                @pl.when(cc + 3 < n_chunks)
                def _():
                    n0, n1 = gather_descs(cc + 3, (b + 3) % 4)
                    n0.start()
                    n1.start()

                @pl.when(cc >= 4)
                def _():
                    out_desc(cc, b).wait()

                @pl.loop(0, G)
                def _tok(t):
                    ga = g_v[pl.ds((cc * G + t) * 16, 16)]
                    gb = g_v[pl.ds((per_w + cc * G + t) * 16, 16)]
                    for j in range(H // 16):
                        sl = pl.ds(j * 16, 16)
                        o_v[b, t, sl] = (r_v[b, t, sl] * ga
                                         + r_v[b, G + t, sl] * gb)

                out_desc(cc, b).start()

        for tail in range(4):
            out_desc(n_chunks - 4 + tail, tail).wait()

    return combine


def kernel(hidden_states, expert_outputs, W1, b1, W2, b2):
    B, S, H = hidden_states.shape
    E = W2.shape[0]
    N = B * S
    P = 2
    Np = N // P
    x = hidden_states.reshape(N, H)
    table = expert_outputs.reshape(E * N, H)
    w1t = W1.T
    b1r = b1.reshape(1, H)
    w2t = W2.T
    b2r = b2.reshape(1, E)
    combine = _make_combine(Np, H, n_total=N)
    outs = []
    for p in range(P):
        xs = lax.slice_in_dim(x, p * Np, (p + 1) * Np, axis=0)
        i0r, i1r, g0r, g1r = _router(xs, w1t, b1r, w2t, b2r, N, p * Np)
        i0 = i0r[:, 0]
        i1 = i1r[:, 0]
        g0 = g0r.reshape(Np * 16)
        g1 = g1r.reshape(Np * 16)
        outs.append(combine(table, i0, i1, g0, g1))
    out = jnp.concatenate(outs, axis=0)
    return out.reshape(B, S, H)


# speedup vs baseline: 1.4024x; 1.4024x over previous
import functools

import jax
import jax.numpy as jnp
from jax import lax
from jax.experimental import pallas as pl
from jax.experimental.pallas import tpu as pltpu
from jax.experimental.pallas import tpu_sc as plsc


def _router_body(T, E, n_total, tok_base, x_ref, w1t_ref, b1_ref, w2t_ref,
                 b2_ref, i0_ref, i1_ref, g0_ref, g1_ref):
    i = pl.program_id(0)
    h = jnp.dot(x_ref[...], w1t_ref[...], preferred_element_type=jnp.float32)
    h = jnp.maximum(h + b1_ref[...], 0.0)
    logits = jnp.dot(h, w2t_ref[...], preferred_element_type=jnp.float32)
    logits = logits + b2_ref[...]
    m = jnp.max(logits, axis=1, keepdims=True)
    p = jnp.exp(logits - m)
    p = p / jnp.sum(p, axis=1, keepdims=True)
    lane = lax.broadcasted_iota(jnp.int32, (T, E), 1)
    p1 = jnp.max(p, axis=1, keepdims=True)
    i1 = jnp.min(jnp.where(p == p1, lane, E), axis=1, keepdims=True)
    pm = jnp.where(lane == i1, -jnp.inf, p)
    p2 = jnp.max(pm, axis=1, keepdims=True)
    i2 = jnp.min(jnp.where(pm == p2, lane, E), axis=1, keepdims=True)
    tok = tok_base + i * T + lax.broadcasted_iota(jnp.int32, (T, 1), 0)
    i0_ref[...] = jnp.broadcast_to(i1 * n_total + tok, (T, 16))
    i1_ref[...] = jnp.broadcast_to(i2 * n_total + tok, (T, 16))
    g0_ref[...] = jnp.broadcast_to(p1, (T, 16))
    g1_ref[...] = jnp.broadcast_to(p2, (T, 16))


def _router(x, w1t, b1, w2t, b2, n_total, tok_base, T=1024):
    N, H = x.shape
    E = w2t.shape[1]
    body = functools.partial(_router_body, T, E, n_total, tok_base)
    grid = (N // T,)
    outs = pl.pallas_call(
        body,
        grid=grid,
        in_specs=[
            pl.BlockSpec((T, H), lambda i: (i, 0)),
            pl.BlockSpec((H, H), lambda i: (0, 0)),
            pl.BlockSpec((1, H), lambda i: (0, 0)),
            pl.BlockSpec((H, E), lambda i: (0, 0)),
            pl.BlockSpec((1, E), lambda i: (0, 0)),
        ],
        out_specs=[
            pl.BlockSpec((T, 16), lambda i: (i, 0)),
            pl.BlockSpec((T, 16), lambda i: (i, 0)),
            pl.BlockSpec((T, 16), lambda i: (i, 0)),
            pl.BlockSpec((T, 16), lambda i: (i, 0)),
        ],
        out_shape=[
            jax.ShapeDtypeStruct((N, 16), jnp.int32),
            jax.ShapeDtypeStruct((N, 16), jnp.int32),
            jax.ShapeDtypeStruct((N, 16), jnp.float32),
            jax.ShapeDtypeStruct((N, 16), jnp.float32),
        ],
    )(x, w1t, b1, w2t, b2)
    return outs


def _make_combine(N, H, G=8, n_total=None):
    n_workers = 32
    per_w = N // n_workers
    n_chunks = per_w // G
    assert n_chunks % 4 == 0
    mesh = plsc.VectorSubcoreMesh(
        core_axis_name="c", subcore_axis_name="s", num_cores=2, num_subcores=16)

    @functools.partial(
        pl.kernel,
        out_type=jax.ShapeDtypeStruct((N, H), jnp.float32),
        mesh=mesh,
        scratch_types=[
            pltpu.VMEM((2 * per_w,), jnp.int32),
            pltpu.VMEM((2 * per_w * 16,), jnp.float32),
            pltpu.VMEM((4, 2 * G, H), jnp.float32),
            pltpu.VMEM((4, G, H), jnp.float32),
            pltpu.SemaphoreType.DMA,
            pltpu.SemaphoreType.DMA,
            pltpu.SemaphoreType.DMA,
            pltpu.SemaphoreType.DMA,
            pltpu.SemaphoreType.DMA,
            pltpu.SemaphoreType.DMA,
            pltpu.SemaphoreType.DMA,
            pltpu.SemaphoreType.DMA,
        ],
    )
    def combine(table, idx_cat, g0, g1, out,
                idx_v, g_v, r_v, o_v,
                sg0, sg1, sg2, sg3, so0, so1, so2, so3):
        wid = lax.axis_index("s") * 2 + lax.axis_index("c")
        wbase = wid * per_w
        sg = (sg0, sg1, sg2, sg3)
        so = (so0, so1, so2, so3)

        pltpu.sync_copy(idx_cat.at[pl.ds(2 * wbase, 2 * per_w)], idx_v)
        pltpu.sync_copy(g0.at[pl.ds(wbase * 16, per_w * 16)],
                        g_v.at[pl.ds(0, per_w * 16)])
        pltpu.sync_copy(g1.at[pl.ds(wbase * 16, per_w * 16)],
                        g_v.at[pl.ds(per_w * 16, per_w * 16)])

        def gather_descs(cc, b):
            d = pltpu.make_async_copy(
                table.at[idx_v.at[pl.ds(2 * G * cc, 2 * G)]],
                r_v.at[b], sg[b])
            return (d,)

        def out_desc(cc, b):
            return pltpu.make_async_copy(
                o_v.at[b], out.at[pl.ds(wbase + cc * G, G)], so[b])

        for pre in range(3):
            for d in gather_descs(pre, pre):
                d.start()

        @pl.loop(0, n_chunks, step=4)
        def _quad(c):
            for b in range(4):
                cc = c + b
                for d in gather_descs(cc, b):
                    d.wait()

                @pl.when(cc + 3 < n_chunks)
                def _():
                    for d in gather_descs(cc + 3, (b + 3) % 4):
                        d.start()

                @pl.when(cc >= 4)
                def _():
                    out_desc(cc, b).wait()

                @pl.loop(0, G)
                def _tok(t):
                    ga = g_v[pl.ds((cc * G + t) * 16, 16)]
                    gb = g_v[pl.ds((per_w + cc * G + t) * 16, 16)]
                    for j in range(H // 16):
                        sl = pl.ds(j * 16, 16)
                        o_v[b, t, sl] = (r_v[b, t, sl] * ga
                                         + r_v[b, G + t, sl] * gb)

                out_desc(cc, b).start()

        for tail in range(4):
            out_desc(n_chunks - 4 + tail, tail).wait()

    return combine


def kernel(hidden_states, expert_outputs, W1, b1, W2, b2):
    B, S, H = hidden_states.shape
    E = W2.shape[0]
    N = B * S
    G = 8
    x = hidden_states.reshape(N, H)
    table = expert_outputs.reshape(E * N, H)
    i0r, i1r, g0r, g1r = _router(
        x, W1.T, b1.reshape(1, H), W2.T, b2.reshape(1, E), N, 0)
    i0 = i0r[:, 0]
    i1 = i1r[:, 0]
    idx_cat = jnp.stack(
        [i0.reshape(N // G, G), i1.reshape(N // G, G)], axis=1).reshape(2 * N)
    g0 = g0r.reshape(N * 16)
    g1 = g1r.reshape(N * 16)
    out = _make_combine(N, H, G=G)(table, idx_cat, g0, g1)
    return out.reshape(B, S, H)


# speedup vs baseline: 1.7958x; 1.2805x over previous
import dataclasses
import functools

import jax
import jax.numpy as jnp
from jax import lax
from jax.experimental import pallas as pl
from jax.experimental.pallas import tpu as pltpu
from jax.experimental.pallas import tpu_sc as plsc


def _router_body(T, E, n_total, x_ref, w1t_ref, b1_ref, w2_ref, b2_ref,
                 i0_ref, i1_ref, g0_ref, g1_ref):
    i = pl.program_id(0)
    h = jnp.dot(x_ref[...], w1t_ref[...], preferred_element_type=jnp.float32)
    h = jnp.maximum(h + b1_ref[...], 0.0)
    logits = lax.dot_general(w2_ref[...], h, (((1,), (1,)), ((), ())),
                             preferred_element_type=jnp.float32)
    logits = logits + b2_ref[...]
    m = jnp.max(logits, axis=0, keepdims=True)
    p = jnp.exp(logits - m)
    p = p / jnp.sum(p, axis=0, keepdims=True)
    sub = lax.broadcasted_iota(jnp.int32, (E, T), 0)
    p1 = jnp.max(p, axis=0, keepdims=True)
    e1 = jnp.min(jnp.where(p == p1, sub, E), axis=0, keepdims=True)
    pm = jnp.where(sub == e1, -jnp.inf, p)
    p2 = jnp.max(pm, axis=0, keepdims=True)
    e2 = jnp.min(jnp.where(pm == p2, sub, E), axis=0, keepdims=True)
    tok = i * T + lax.broadcasted_iota(jnp.int32, (1, T), 1)
    i0_ref[0] = e1 * n_total + tok
    i1_ref[0] = e2 * n_total + tok
    g0_ref[0] = p1
    g1_ref[0] = p2


def _router(x, w1t, b1, w2, b2, T=1024):
    N, H = x.shape
    E = w2.shape[0]
    nb = N // T
    body = functools.partial(_router_body, T, E, N)
    outs = pl.pallas_call(
        body,
        grid=(nb,),
        in_specs=[
            pl.BlockSpec((T, H), lambda i: (i, 0)),
            pl.BlockSpec((H, H), lambda i: (0, 0)),
            pl.BlockSpec((1, H), lambda i: (0, 0)),
            pl.BlockSpec((E, H), lambda i: (0, 0)),
            pl.BlockSpec((E, 1), lambda i: (0, 0)),
        ],
        out_specs=[
            pl.BlockSpec((1, 1, T), lambda i: (i, 0, 0)),
            pl.BlockSpec((1, 1, T), lambda i: (i, 0, 0)),
            pl.BlockSpec((1, 1, T), lambda i: (i, 0, 0)),
            pl.BlockSpec((1, 1, T), lambda i: (i, 0, 0)),
        ],
        out_shape=[
            jax.ShapeDtypeStruct((nb, 1, T), jnp.int32),
            jax.ShapeDtypeStruct((nb, 1, T), jnp.int32),
            jax.ShapeDtypeStruct((nb, 1, T), jnp.float32),
            jax.ShapeDtypeStruct((nb, 1, T), jnp.float32),
        ],
    )(x, w1t, b1, w2, b2)
    return outs


def _make_combine(N, H, G=8):
    n_workers = 32
    per_w = N // n_workers
    n_chunks = per_w // G
    assert n_chunks % 4 == 0
    mesh = plsc.VectorSubcoreMesh(
        core_axis_name="c", subcore_axis_name="s", num_cores=2, num_subcores=16)

    cp = pltpu.CompilerParams()
    if "needs_layout_passes" in pltpu.CompilerParams.__dataclass_fields__:
        cp = dataclasses.replace(cp, needs_layout_passes=False)

    @functools.partial(
        pl.kernel,
        out_type=jax.ShapeDtypeStruct((N, H), jnp.float32),
        mesh=mesh,
        compiler_params=cp,
        scratch_types=[
            pltpu.VMEM((2 * per_w,), jnp.int32),
            pltpu.VMEM((2 * per_w,), jnp.float32),
            pltpu.VMEM((4, 2 * G, H), jnp.float32),
            pltpu.VMEM((4, G, H), jnp.float32),
            pltpu.SemaphoreType.DMA,
            pltpu.SemaphoreType.DMA,
            pltpu.SemaphoreType.DMA,
            pltpu.SemaphoreType.DMA,
            pltpu.SemaphoreType.DMA,
            pltpu.SemaphoreType.DMA,
            pltpu.SemaphoreType.DMA,
            pltpu.SemaphoreType.DMA,
        ],
    )
    def combine(table, i0, i1, g0, g1, out,
                idx_v, g_v, r_v, o_v,
                sg0, sg1, sg2, sg3, so0, so1, so2, so3):
        wid = lax.axis_index("s") * 2 + lax.axis_index("c")
        wbase = wid * per_w
        sg = (sg0, sg1, sg2, sg3)
        so = (so0, so1, so2, so3)

        pltpu.sync_copy(i0.at[pl.ds(wbase, per_w)], idx_v.at[pl.ds(0, per_w)])
        pltpu.sync_copy(i1.at[pl.ds(wbase, per_w)],
                        idx_v.at[pl.ds(per_w, per_w)])
        pltpu.sync_copy(g0.at[pl.ds(wbase, per_w)], g_v.at[pl.ds(0, per_w)])
        pltpu.sync_copy(g1.at[pl.ds(wbase, per_w)],
                        g_v.at[pl.ds(per_w, per_w)])

        def gather_descs(cc, b):
            base = cc * G
            d0 = pltpu.make_async_copy(
                table.at[idx_v.at[pl.ds(base, G)]],
                r_v.at[b, pl.ds(0, G)], sg[b])
            d1 = pltpu.make_async_copy(
                table.at[idx_v.at[pl.ds(per_w + base, G)]],
                r_v.at[b, pl.ds(G, G)], sg[b])
            return (d0, d1)

        def out_desc(cc, b):
            return pltpu.make_async_copy(
                o_v.at[b], out.at[pl.ds(wbase + cc * G, G)], so[b])

        for pre in range(3):
            for d in gather_descs(pre, pre):
                d.start()

        @pl.loop(0, n_chunks, step=4)
        def _quad(c):
            for b in range(4):
                cc = c + b
                for d in gather_descs(cc, b):
                    d.wait()

                @pl.when(cc + 3 < n_chunks)
                def _():
                    for d in gather_descs(cc + 3, (b + 3) % 4):
                        d.start()

                @pl.when(cc >= 4)
                def _():
                    out_desc(cc, b).wait()

                @pl.loop(0, G)
                def _tok(t):
                    ia = jnp.full((16,), cc * G + t, jnp.int32)
                    ga = plsc.load_gather(g_v, [ia])
                    gb = plsc.load_gather(g_v, [ia + per_w])
                    for j in range(H // 16):
                        sl = pl.ds(j * 16, 16)
                        o_v[b, t, sl] = (r_v[b, t, sl] * ga
                                         + r_v[b, G + t, sl] * gb)

                out_desc(cc, b).start()

        for tail in range(4):
            out_desc(n_chunks - 4 + tail, tail).wait()

    return combine


def kernel(hidden_states, expert_outputs, W1, b1, W2, b2):
    B, S, H = hidden_states.shape
    E = W2.shape[0]
    N = B * S
    x = hidden_states.reshape(N, H)
    table = expert_outputs.reshape(E * N, H)
    i0r, i1r, g0r, g1r = _router(
        x, W1.T, b1.reshape(1, H), W2, b2.reshape(E, 1))
    i0 = i0r.reshape(N)
    i1 = i1r.reshape(N)
    g0 = g0r.reshape(N)
    g1 = g1r.reshape(N)
    out = _make_combine(N, H)(table, i0, i1, g0, g1)
    return out.reshape(B, S, H)
